# baseline (speedup 1.0000x reference)
import functools

import jax
import jax.numpy as jnp
from jax import lax
from jax.experimental import pallas as pl
from jax.experimental.pallas import tpu as pltpu
from jax.experimental.pallas import tpu_sc as plsc

VOCAB = 1000000
DIM = 64
BATCH = 4096
HIST = 200
HALF = VOCAB // 2

NC = 2
NS = 16
NW = NC * NS
L = 16

CB = 128
NBC = BATCH // CB
NITEM = HIST * NBC
ITEMS_PER_W = NITEM // NW

PACK_W = 2048
PACK_GRID = 245
HALF2 = PACK_GRID * PACK_W


def _pack_body(a_ref, o_ref):
    o_ref[:, :DIM] = a_ref[:, :PACK_W].T
    o_ref[:, DIM:] = a_ref[:, PACK_W:].T


_pack_kernel = pl.pallas_call(
    _pack_body,
    out_shape=jax.ShapeDtypeStruct((HALF2, 2 * DIM), jnp.float32),
    grid=(PACK_GRID,),
    in_specs=[pl.BlockSpec((DIM, 2 * PACK_W), lambda j: (0, j))],
    out_specs=pl.BlockSpec((PACK_W, 2 * DIM), lambda j: (j, 0)),
)


@functools.partial(
    pl.kernel,
    out_type=jax.ShapeDtypeStruct((HIST, DIM, BATCH), jnp.float32),
    mesh=plsc.VectorSubcoreMesh(
        core_axis_name="c", subcore_axis_name="s", num_cores=NC, num_subcores=NS
    ),
    scratch_types=[
        *[pltpu.VMEM((CB,), jnp.int32) for _ in range(4)],
        *[pltpu.VMEM((CB,), jnp.int32) for _ in range(2)],
        *[pltpu.VMEM((CB, 2 * DIM + 8), jnp.float32) for _ in range(2)],
        *[pltpu.VMEM((DIM, CB), jnp.float32) for _ in range(2)],
        *[pltpu.SemaphoreType.DMA for _ in range(8)],
    ],
    compiler_params=pltpu.CompilerParams(needs_layout_passes=False),
)
def _gather_kernel(idx_hbm, table2_hbm, out_hbm, *bufs):
    idx_v = list(bufs[0:4])
    pidx_v = list(bufs[4:6])
    pair_v = list(bufs[6:8])
    tr_v = list(bufs[8:10])
    isem = list(bufs[10:14])
    gsem = list(bufs[14:16])
    osem = list(bufs[16:18])

    wid = lax.axis_index("s") * NC + lax.axis_index("c")
    q0 = wid * ITEMS_PER_W

    def idx_slice(i):
        q = q0 + i
        return idx_hbm.at[q // NBC, pl.ds((q % NBC) * CB, CB)]

    def start_idx(i, r):
        pltpu.async_copy(idx_slice(i), idx_v[r], isem[r])

    def wait_idx(i, r):
        pltpu.make_async_copy(idx_slice(i), idx_v[r], isem[r]).wait()

    def mk_pidx(r, b):
        def grp(g, c2):
            v = idx_v[r][pl.ds(g * L, L)]
            j = lax.shift_right_logical(v, 12)
            q = jnp.bitwise_and(v, 4095)
            p = lax.shift_left(j, 11) + jnp.bitwise_and(q, 2047)
            pidx_v[b][pl.ds(g * L, L)] = p
            return c2

        lax.fori_loop(0, CB // L, grp, 0)

    def start_gather(b):
        pltpu.async_copy(
            table2_hbm.at[pidx_v[b]], pair_v[b].at[:, pl.ds(0, 2 * DIM)], gsem[b]
        )

    def wait_gather(b):
        pltpu.make_async_copy(
            table2_hbm.at[pidx_v[b]], pair_v[b].at[:, pl.ds(0, 2 * DIM)], gsem[b]
        ).wait()

    def out_slice(i):
        q = q0 + i
        return out_hbm.at[q // NBC, :, pl.ds((q % NBC) * CB, CB)]

    def start_store(i, b):
        pltpu.async_copy(tr_v[b], out_slice(i), osem[b])

    def wait_store(i, b):
        pltpu.make_async_copy(tr_v[b], out_slice(i), osem[b]).wait()

    def transpose(r, b):
        def grp(g, c2):
            jvec = g * L + lax.iota(jnp.int32, L)
            raw = idx_v[r][pl.ds(g * L, L)]
            hi = jnp.bitwise_and(raw, 4095) >= PACK_W
            colbase = jnp.where(hi, DIM, 0)
            for c4 in range(0, DIM, 4):
                v0 = plsc.load_gather(pair_v[b], [jvec, colbase + c4])
                v1 = plsc.load_gather(pair_v[b], [jvec, colbase + (c4 + 1)])
                v2 = plsc.load_gather(pair_v[b], [jvec, colbase + (c4 + 2)])
                v3 = plsc.load_gather(pair_v[b], [jvec, colbase + (c4 + 3)])
                tr_v[b][c4, pl.ds(g * L, L)] = v0
                tr_v[b][c4 + 1, pl.ds(g * L, L)] = v1
                tr_v[b][c4 + 2, pl.ds(g * L, L)] = v2
                tr_v[b][c4 + 3, pl.ds(g * L, L)] = v3
            return c2

        lax.fori_loop(0, CB // L, grp, 0)

    start_idx(0, 0)
    start_idx(1, 1)
    wait_idx(0, 0)
    mk_pidx(0, 0)
    start_gather(0)

    def outer(h, carry):
        for u in range(4):
            i = h * 4 + u
            b = u % 2
            bn = 1 - b
            rn = (u + 1) % 4
            wait_gather(b)

            @pl.when(i + 1 < ITEMS_PER_W)
            def _():
                wait_idx(i + 1, rn)
                mk_pidx(rn, bn)

                @pl.when(i + 1 >= 2)
                def _():
                    wait_store(i - 1, bn)

                start_gather(bn)

                @pl.when(i + 2 < ITEMS_PER_W)
                def _():
                    start_idx(i + 2, (u + 2) % 4)

            transpose(u % 4, b)
            start_store(i, b)
        return carry

    lax.fori_loop(0, ITEMS_PER_W // 4, outer, 0)

    wait_store(ITEMS_PER_W - 2, 0)
    wait_store(ITEMS_PER_W - 1, 1)


def kernel(inputs, embed_weight):
    idx_t = inputs.T
    table2 = _pack_kernel(embed_weight.T)
    out_t = _gather_kernel(idx_t, table2)
    return jnp.transpose(out_t, (2, 0, 1))


# speedup vs baseline: 1.0190x; 1.0190x over previous
import functools

import jax
import jax.numpy as jnp
from jax import lax
from jax.experimental import pallas as pl
from jax.experimental.pallas import tpu as pltpu
from jax.experimental.pallas import tpu_sc as plsc

VOCAB = 1000000
DIM = 64
BATCH = 4096
HIST = 200
HALF = VOCAB // 2

NC = 2
NS = 16
NW = NC * NS
L = 16

CB = 256
NBC = BATCH // CB
NITEM = HIST * NBC
ITEMS_PER_W = NITEM // NW

PACK_W = 2048
PACK_GRID = 245
HALF2 = PACK_GRID * PACK_W


def _pack_body(a_ref, o_ref):
    o_ref[:, :DIM] = a_ref[:, :PACK_W].T
    o_ref[:, DIM:] = a_ref[:, PACK_W:].T


_pack_kernel = pl.pallas_call(
    _pack_body,
    out_shape=jax.ShapeDtypeStruct((HALF2, 2 * DIM), jnp.float32),
    grid=(PACK_GRID,),
    in_specs=[pl.BlockSpec((DIM, 2 * PACK_W), lambda j: (0, j))],
    out_specs=pl.BlockSpec((PACK_W, 2 * DIM), lambda j: (j, 0)),
)


@functools.partial(
    pl.kernel,
    out_type=jax.ShapeDtypeStruct((HIST, DIM, BATCH), jnp.float32),
    mesh=plsc.VectorSubcoreMesh(
        core_axis_name="c", subcore_axis_name="s", num_cores=NC, num_subcores=NS
    ),
    scratch_types=[
        *[pltpu.VMEM((CB,), jnp.int32) for _ in range(4)],
        *[pltpu.VMEM((CB,), jnp.int32) for _ in range(2)],
        *[pltpu.VMEM((CB, 2 * DIM), jnp.float32) for _ in range(2)],
        *[pltpu.VMEM((DIM, CB), jnp.float32) for _ in range(2)],
        *[pltpu.SemaphoreType.DMA for _ in range(8)],
    ],
    compiler_params=pltpu.CompilerParams(needs_layout_passes=False),
)
def _gather_kernel(idx_hbm, table2_hbm, out_hbm, *bufs):
    idx_v = list(bufs[0:4])
    pidx_v = list(bufs[4:6])
    pair_v = list(bufs[6:8])
    tr_v = list(bufs[8:10])
    isem = list(bufs[10:14])
    gsem = list(bufs[14:16])
    osem = list(bufs[16:18])

    wid = lax.axis_index("s") * NC + lax.axis_index("c")
    q0 = wid * ITEMS_PER_W

    def idx_slice(i):
        q = q0 + i
        return idx_hbm.at[q // NBC, pl.ds((q % NBC) * CB, CB)]

    def start_idx(i, r):
        pltpu.async_copy(idx_slice(i), idx_v[r], isem[r])

    def wait_idx(i, r):
        pltpu.make_async_copy(idx_slice(i), idx_v[r], isem[r]).wait()

    def mk_pidx(r, b):
        def grp(g, c2):
            v = idx_v[r][pl.ds(g * L, L)]
            j = lax.shift_right_logical(v, 12)
            q = jnp.bitwise_and(v, 4095)
            p = lax.shift_left(j, 11) + jnp.bitwise_and(q, 2047)
            pidx_v[b][pl.ds(g * L, L)] = p
            return c2

        lax.fori_loop(0, CB // L, grp, 0)

    def start_gather(b):
        pltpu.async_copy(table2_hbm.at[pidx_v[b]], pair_v[b], gsem[b])

    def wait_gather(b):
        pltpu.make_async_copy(table2_hbm.at[pidx_v[b]], pair_v[b], gsem[b]).wait()

    def out_slice(i):
        q = q0 + i
        return out_hbm.at[q // NBC, :, pl.ds((q % NBC) * CB, CB)]

    def start_store(i, b):
        pltpu.async_copy(tr_v[b], out_slice(i), osem[b])

    def wait_store(i, b):
        pltpu.make_async_copy(tr_v[b], out_slice(i), osem[b]).wait()

    def transpose(r, b):
        def grp(g, c2):
            jvec = g * L + lax.iota(jnp.int32, L)
            raw = idx_v[r][pl.ds(g * L, L)]
            hi = jnp.bitwise_and(raw, 4095) >= PACK_W
            colbase = jnp.where(hi, DIM, 0)
            for c4 in range(0, DIM, 4):
                v0 = plsc.load_gather(pair_v[b], [jvec, colbase + c4])
                v1 = plsc.load_gather(pair_v[b], [jvec, colbase + (c4 + 1)])
                v2 = plsc.load_gather(pair_v[b], [jvec, colbase + (c4 + 2)])
                v3 = plsc.load_gather(pair_v[b], [jvec, colbase + (c4 + 3)])
                tr_v[b][c4, pl.ds(g * L, L)] = v0
                tr_v[b][c4 + 1, pl.ds(g * L, L)] = v1
                tr_v[b][c4 + 2, pl.ds(g * L, L)] = v2
                tr_v[b][c4 + 3, pl.ds(g * L, L)] = v3
            return c2

        lax.fori_loop(0, CB // L, grp, 0)

    start_idx(0, 0)
    start_idx(1, 1)
    wait_idx(0, 0)
    mk_pidx(0, 0)
    start_gather(0)

    def outer(h, carry):
        for u in range(4):
            i = h * 4 + u
            b = u % 2
            bn = 1 - b
            rn = (u + 1) % 4
            wait_gather(b)

            @pl.when(i + 1 < ITEMS_PER_W)
            def _():
                wait_idx(i + 1, rn)
                mk_pidx(rn, bn)

                @pl.when(i + 1 >= 2)
                def _():
                    wait_store(i - 1, bn)

                start_gather(bn)

                @pl.when(i + 2 < ITEMS_PER_W)
                def _():
                    start_idx(i + 2, (u + 2) % 4)

            transpose(u % 4, b)
            start_store(i, b)
        return carry

    lax.fori_loop(0, ITEMS_PER_W // 4, outer, 0)

    wait_store(ITEMS_PER_W - 2, 0)
    wait_store(ITEMS_PER_W - 1, 1)


def kernel(inputs, embed_weight):
    idx_t = inputs.T
    table2 = _pack_kernel(embed_weight.T)
    out_t = _gather_kernel(idx_t, table2)
    return jnp.transpose(out_t, (2, 0, 1))
